# baseline (speedup 1.0000x reference)
import jax
import jax.numpy as jnp
from jax import lax
from jax.experimental import pallas as pl
from jax.experimental.pallas import tpu as pltpu
from jax.experimental.pallas import tpu_sc as plsc

_N = 10000
_E = 320000
_F = 128
_H = 512
_G = 64
_C = 40

_NC = 2
_NS = 16
_NW = _NC * _NS
_L = 16

_K = 128
_NCH = 160
_NQ = _NCH // 2
_EPC = _NCH * _K
_EP = _EPC * _NS
_NP = 10240
_ZR = 128
_RPC = _NP // _ZR // _NS


def _sc_body(xt_hbm, srct_hbm, dst_hbm, zero_hbm, agg_out,
             srci0, srci1, dsti0, dsti1, rows0, rows1, agg_sh,
             semg0, semg1, semi0, semi1):
    cid = lax.axis_index("c")
    sid = lax.axis_index("s")

    def _idx_load(c, srcb, dstb):
        base = pl.multiple_of(sid * _EPC + c * _K, 128)
        pltpu.sync_copy(srct_hbm.at[cid].at[pl.ds(base, _K)], srcb)
        pltpu.sync_copy(dst_hbm.at[pl.ds(base, _K)], dstb)

    def _gather_start(srcb, buf, sem):
        pltpu.async_copy(xt_hbm.at[srcb], buf, sem)

    def _gather_wait(srcb, buf, sem):
        pltpu.make_async_copy(xt_hbm.at[srcb], buf, sem).wait()

    def _scatter(dstb, buf):
        pltpu.sync_copy(buf, agg_sh.at[dstb], add=True)

    pltpu.sync_copy(zero_hbm, rows0)

    def zero_region(k, carry):
        ro = pl.multiple_of((sid * _RPC + k) * _ZR, 8)
        pltpu.sync_copy(rows0, agg_sh.at[pl.ds(ro, _ZR)])
        return carry

    lax.fori_loop(0, _RPC, zero_region, 0)
    plsc.subcore_barrier()

    def chunk(c, carry):
        _idx_load(c, srci0, dsti0)
        pltpu.async_copy(xt_hbm.at[srci0], rows0, semg0).wait()
        _scatter(dsti0, rows0)
        return carry

    lax.fori_loop(0, _NCH, chunk, 0)
    plsc.subcore_barrier()

    def out_region(k, carry):
        ro = pl.multiple_of((sid * _RPC + k) * _ZR, 8)
        pltpu.sync_copy(agg_sh.at[pl.ds(ro, _ZR)],
                        agg_out.at[cid].at[pl.ds(ro, _ZR)])
        return carry

    lax.fori_loop(0, _RPC, out_region, 0)


def _build_sc_aggregate():
    mesh = plsc.VectorSubcoreMesh(
        core_axis_name="c", subcore_axis_name="s",
        num_cores=_NC, num_subcores=_NS)
    return pl.kernel(
        _sc_body,
        out_type=jax.ShapeDtypeStruct((_NC, _NP, _F), jnp.float32),
        mesh=mesh,
        scratch_types=[
            pltpu.VMEM((_K,), jnp.int32),
            pltpu.VMEM((_K,), jnp.int32),
            pltpu.VMEM((_K,), jnp.int32),
            pltpu.VMEM((_K,), jnp.int32),
            pltpu.VMEM((_K, _F), jnp.float32),
            pltpu.VMEM((_K, _F), jnp.float32),
            pltpu.VMEM_SHARED((_NP, _F), jnp.float32),
            pltpu.SemaphoreType.DMA,
            pltpu.SemaphoreType.DMA,
            pltpu.SemaphoreType.DMA,
            pltpu.SemaphoreType.DMA,
        ],
    )


def _sc_aggregate(x, src, dst):
    xt = jnp.concatenate(
        [x, jnp.ones_like(x), jnp.zeros((8, _F), jnp.float32)], axis=0)
    pad = jnp.full((_EP - _E,), 2 * _N, jnp.int32)
    srct = jnp.stack([jnp.concatenate([src, pad]),
                      jnp.concatenate([src + _N, pad])])
    dstp = jnp.concatenate([dst, jnp.full((_EP - _E,), _N, jnp.int32)])
    zero_blk = jnp.zeros((_ZR, _F), jnp.float32)
    return _build_sc_aggregate()(xt, srct, dstp, zero_blk)


_BR = 400
_NB = _N // _BR


def _tc_a_body(a0, d0, x, wl, wr, bl, h_out, s1, s2):
    i = pl.program_id(0)
    d = jnp.maximum(d0[...], 1.0)
    mean = a0[...] / d
    h = (jnp.dot(mean, wl[...], preferred_element_type=jnp.float32)
         + jnp.dot(x[...], wr[...], preferred_element_type=jnp.float32)
         + bl[...])
    h_out[...] = h

    @pl.when(i == 0)
    def _():
        s1[...] = jnp.zeros_like(s1)
        s2[...] = jnp.zeros_like(s2)

    s1[...] += jnp.sum(h, axis=0, keepdims=True)
    s2[...] += jnp.sum(h * h, axis=0, keepdims=True)


def _tc_b_body(h_in, b_in, s1, s2, g1, be1, gmax, gsum, gcnt):
    i = pl.program_id(0)
    mu = s1[...] * (1.0 / _N)
    var = s2[...] * (1.0 / _N) - mu * mu
    rstd = lax.rsqrt(var + 1e-5)
    h = (h_in[...] - mu) * rstd * g1[...] + be1[...]
    h = jnp.maximum(h, 0.0)

    bcol = b_in[...]
    iota_g = lax.broadcasted_iota(jnp.int32, (1, _G), 1)
    oh = (bcol == iota_g).astype(jnp.float32)

    @pl.when(i == 0)
    def _():
        gmax[...] = jnp.full_like(gmax, -1e30)
        gsum[...] = jnp.zeros_like(gsum)
        gcnt[...] = jnp.zeros_like(gcnt)

    gsum[...] += lax.dot_general(oh, h, (((0,), (0,)), ((), ())),
                                 preferred_element_type=jnp.float32)
    ones_col = jnp.ones((_BR, 1), jnp.float32)
    gcnt[...] += lax.dot_general(oh, ones_col, (((0,), (0,)), ((), ())),
                                 preferred_element_type=jnp.float32)

    glo = jnp.min(bcol)
    ghi = jnp.max(bcol)
    row_ids = lax.broadcasted_iota(jnp.int32, (_G, 1), 0)
    acc = gmax[...]

    def mbody(g, acc):
        m = jnp.max(jnp.where(bcol == g, h, -1e30), axis=0, keepdims=True)
        return jnp.where(row_ids == g, jnp.maximum(acc, m), acc)

    gmax[...] = lax.fori_loop(glo, ghi + 1, mbody, acc)


def _tc_c_body(gmax, gsum, gcnt, w1a, w1b, b1, g4, be4, w2, b2, out):
    mx = gmax[...]
    mx = jnp.where(mx > -1e29, mx, 0.0)
    mean = gsum[...] / jnp.maximum(gcnt[...], 1.0)
    z = (jnp.dot(mx, w1a[...], preferred_element_type=jnp.float32)
         + jnp.dot(mean, w1b[...], preferred_element_type=jnp.float32)
         + b1[...])
    mu = jnp.mean(z, axis=0, keepdims=True)
    var = jnp.mean(z * z, axis=0, keepdims=True) - mu * mu
    z = (z - mu) * lax.rsqrt(var + 1e-5) * g4[...] + be4[...]
    z = jnp.maximum(z, 0.0)
    out[...] = jnp.dot(z, w2[...], preferred_element_type=jnp.float32) + b2[...]


def kernel(x, edge_index, batch, W_l, b_l, W_r, gamma1, beta1,
           fc1_W, fc1_b, gamma4, beta4, fc2_W, fc2_b):
    src = edge_index[0]
    dst = edge_index[1]

    aggdeg = _sc_aggregate(x, src, dst)
    a0 = aggdeg[0, :_N]
    d0 = aggdeg[1, :_N, 0:1]

    row_spec = pl.BlockSpec((_BR, _F), lambda i: (i, 0))
    col1_spec = pl.BlockSpec((_BR, 1), lambda i: (i, 0))
    w_spec = pl.BlockSpec((_F, _H), lambda i: (0, 0))
    vec_spec = pl.BlockSpec((1, _H), lambda i: (0, 0))

    h_raw, s1, s2 = pl.pallas_call(
        _tc_a_body,
        grid=(_NB,),
        in_specs=[row_spec, col1_spec, row_spec,
                  w_spec, w_spec, vec_spec],
        out_specs=[pl.BlockSpec((_BR, _H), lambda i: (i, 0)),
                   vec_spec, vec_spec],
        out_shape=[jax.ShapeDtypeStruct((_N, _H), jnp.float32),
                   jax.ShapeDtypeStruct((1, _H), jnp.float32),
                   jax.ShapeDtypeStruct((1, _H), jnp.float32)],
    )(a0, d0, x, W_l, W_r, b_l[None, :])

    gmax, gsum, gcnt = pl.pallas_call(
        _tc_b_body,
        grid=(_NB,),
        in_specs=[pl.BlockSpec((_BR, _H), lambda i: (i, 0)),
                  col1_spec, vec_spec, vec_spec, vec_spec, vec_spec],
        out_specs=[pl.BlockSpec((_G, _H), lambda i: (0, 0)),
                   pl.BlockSpec((_G, _H), lambda i: (0, 0)),
                   pl.BlockSpec((_G, 1), lambda i: (0, 0))],
        out_shape=[jax.ShapeDtypeStruct((_G, _H), jnp.float32),
                   jax.ShapeDtypeStruct((_G, _H), jnp.float32),
                   jax.ShapeDtypeStruct((_G, 1), jnp.float32)],
    )(h_raw, batch[:, None], s1, s2, gamma1[None, :], beta1[None, :])

    logits = pl.pallas_call(
        _tc_c_body,
        in_specs=[pl.BlockSpec((_G, _H), lambda: (0, 0)),
                  pl.BlockSpec((_G, _H), lambda: (0, 0)),
                  pl.BlockSpec((_G, 1), lambda: (0, 0)),
                  pl.BlockSpec((_H, _H), lambda: (0, 0)),
                  pl.BlockSpec((_H, _H), lambda: (0, 0)),
                  pl.BlockSpec((1, _H), lambda: (0, 0)),
                  pl.BlockSpec((1, _H), lambda: (0, 0)),
                  pl.BlockSpec((1, _H), lambda: (0, 0)),
                  pl.BlockSpec((_H, _C), lambda: (0, 0)),
                  pl.BlockSpec((1, _C), lambda: (0, 0))],
        out_specs=pl.BlockSpec((_G, _C), lambda: (0, 0)),
        out_shape=jax.ShapeDtypeStruct((_G, _C), jnp.float32),
    )(gmax, gsum, gcnt, fc1_W[:_H], fc1_W[_H:], fc1_b[None, :],
      gamma4[None, :], beta4[None, :], fc2_W, fc2_b[None, :])

    return logits


# speedup vs baseline: 1.5668x; 1.5668x over previous
import jax
import jax.numpy as jnp
from jax import lax
from jax.experimental import pallas as pl
from jax.experimental.pallas import tpu as pltpu
from jax.experimental.pallas import tpu_sc as plsc

_N = 10000
_E = 320000
_F = 128
_H = 512
_G = 64
_C = 40

_NC = 2
_NS = 16
_NW = _NC * _NS
_L = 16

_K = 128
_NCH = 157
_EPC = _NCH * _K
_EP = _EPC * _NS
_NP = 10240
_ZR = 80
_RPC = _NP // _ZR // _NS


def _sc_body(xt_hbm, srct_hbm, dst_hbm, zero_hbm, agg_out,
             srcbuf, dstbuf, rows, zrow, agg_sh, gsem):
    cid = lax.axis_index("c")
    sid = lax.axis_index("s")

    pltpu.sync_copy(zero_hbm, zrow)

    def zero_region(k, carry):
        ro = pl.multiple_of((sid * _RPC + k) * _ZR, 8)
        pltpu.sync_copy(zrow, agg_sh.at[pl.ds(ro, _ZR)])
        return carry

    lax.fori_loop(0, _RPC, zero_region, 0)
    plsc.subcore_barrier()

    def chunk(c, carry):
        base = pl.multiple_of(sid * _EPC + c * _K, 128)
        pltpu.sync_copy(srct_hbm.at[cid].at[pl.ds(base, _K)], srcbuf)
        pltpu.sync_copy(dst_hbm.at[pl.ds(base, _K)], dstbuf)
        pltpu.async_copy(xt_hbm.at[srcbuf], rows, gsem).wait()
        pltpu.sync_copy(rows, agg_sh.at[dstbuf], add=True)
        return carry

    lax.fori_loop(0, _NCH, chunk, 0)
    plsc.subcore_barrier()

    def out_region(k, carry):
        ro = pl.multiple_of((sid * _RPC + k) * _ZR, 8)
        pltpu.sync_copy(agg_sh.at[pl.ds(ro, _ZR)],
                        agg_out.at[cid].at[pl.ds(ro, _ZR)])
        return carry

    lax.fori_loop(0, _RPC, out_region, 0)


def _build_sc_aggregate():
    mesh = plsc.VectorSubcoreMesh(
        core_axis_name="c", subcore_axis_name="s",
        num_cores=_NC, num_subcores=_NS)
    return pl.kernel(
        _sc_body,
        out_type=jax.ShapeDtypeStruct((_NC, _NP, _F), jnp.float32),
        mesh=mesh,
        scratch_types=[
            pltpu.VMEM((_K,), jnp.int32),
            pltpu.VMEM((_K,), jnp.int32),
            pltpu.VMEM((_K, _F), jnp.float32),
            pltpu.VMEM((_ZR, _F), jnp.float32),
            pltpu.VMEM_SHARED((_NP, _F), jnp.float32),
            pltpu.SemaphoreType.DMA,
        ],
    )


def _sc_aggregate(x, src, dst):
    xt = jnp.concatenate(
        [x, jnp.ones_like(x), jnp.zeros((8, _F), jnp.float32)], axis=0)
    pad = jnp.full((_EP - _E,), 2 * _N, jnp.int32)
    srct = jnp.stack([jnp.concatenate([src, pad]),
                      jnp.concatenate([src + _N, pad])])
    dstp = jnp.concatenate([dst, jnp.full((_EP - _E,), _N, jnp.int32)])
    zero_blk = jnp.zeros((_ZR, _F), jnp.float32)
    return _build_sc_aggregate()(xt, srct, dstp, zero_blk)


_BR = 400
_NB = _N // _BR


def _tc_a_body(a0, d0, x, wl, wr, bl, h_out, s1, s2):
    i = pl.program_id(0)
    d = jnp.maximum(d0[...], 1.0)
    mean = a0[...] / d
    h = (jnp.dot(mean, wl[...], preferred_element_type=jnp.float32)
         + jnp.dot(x[...], wr[...], preferred_element_type=jnp.float32)
         + bl[...])
    h_out[...] = h

    @pl.when(i == 0)
    def _():
        s1[...] = jnp.zeros_like(s1)
        s2[...] = jnp.zeros_like(s2)

    s1[...] += jnp.sum(h, axis=0, keepdims=True)
    s2[...] += jnp.sum(h * h, axis=0, keepdims=True)


def _tc_b_body(h_in, b_in, s1, s2, g1, be1, gmax, gsum, gcnt):
    i = pl.program_id(0)
    mu = s1[...] * (1.0 / _N)
    var = s2[...] * (1.0 / _N) - mu * mu
    rstd = lax.rsqrt(var + 1e-5)
    h = (h_in[...] - mu) * rstd * g1[...] + be1[...]
    h = jnp.maximum(h, 0.0)

    bcol = b_in[...]
    iota_g = lax.broadcasted_iota(jnp.int32, (1, _G), 1)
    oh = (bcol == iota_g).astype(jnp.float32)

    @pl.when(i == 0)
    def _():
        gmax[...] = jnp.full_like(gmax, -1e30)
        gsum[...] = jnp.zeros_like(gsum)
        gcnt[...] = jnp.zeros_like(gcnt)

    gsum[...] += lax.dot_general(oh, h, (((0,), (0,)), ((), ())),
                                 preferred_element_type=jnp.float32)
    ones_col = jnp.ones((_BR, 1), jnp.float32)
    gcnt[...] += lax.dot_general(oh, ones_col, (((0,), (0,)), ((), ())),
                                 preferred_element_type=jnp.float32)

    glo = jnp.min(bcol)
    ghi = jnp.max(bcol)
    row_ids = lax.broadcasted_iota(jnp.int32, (_G, 1), 0)
    acc = gmax[...]

    def mbody(g, acc):
        m = jnp.max(jnp.where(bcol == g, h, -1e30), axis=0, keepdims=True)
        return jnp.where(row_ids == g, jnp.maximum(acc, m), acc)

    gmax[...] = lax.fori_loop(glo, ghi + 1, mbody, acc)


def _tc_c_body(gmax, gsum, gcnt, w1a, w1b, b1, g4, be4, w2, b2, out):
    mx = gmax[...]
    mx = jnp.where(mx > -1e29, mx, 0.0)
    mean = gsum[...] / jnp.maximum(gcnt[...], 1.0)
    z = (jnp.dot(mx, w1a[...], preferred_element_type=jnp.float32)
         + jnp.dot(mean, w1b[...], preferred_element_type=jnp.float32)
         + b1[...])
    mu = jnp.mean(z, axis=0, keepdims=True)
    var = jnp.mean(z * z, axis=0, keepdims=True) - mu * mu
    z = (z - mu) * lax.rsqrt(var + 1e-5) * g4[...] + be4[...]
    z = jnp.maximum(z, 0.0)
    out[...] = jnp.dot(z, w2[...], preferred_element_type=jnp.float32) + b2[...]


def kernel(x, edge_index, batch, W_l, b_l, W_r, gamma1, beta1,
           fc1_W, fc1_b, gamma4, beta4, fc2_W, fc2_b):
    src = edge_index[0]
    dst = edge_index[1]

    aggdeg = _sc_aggregate(x, src, dst)
    a0 = aggdeg[0, :_N]
    d0 = aggdeg[1, :_N, 0:1]

    row_spec = pl.BlockSpec((_BR, _F), lambda i: (i, 0))
    col1_spec = pl.BlockSpec((_BR, 1), lambda i: (i, 0))
    w_spec = pl.BlockSpec((_F, _H), lambda i: (0, 0))
    vec_spec = pl.BlockSpec((1, _H), lambda i: (0, 0))

    h_raw, s1, s2 = pl.pallas_call(
        _tc_a_body,
        grid=(_NB,),
        in_specs=[row_spec, col1_spec, row_spec,
                  w_spec, w_spec, vec_spec],
        out_specs=[pl.BlockSpec((_BR, _H), lambda i: (i, 0)),
                   vec_spec, vec_spec],
        out_shape=[jax.ShapeDtypeStruct((_N, _H), jnp.float32),
                   jax.ShapeDtypeStruct((1, _H), jnp.float32),
                   jax.ShapeDtypeStruct((1, _H), jnp.float32)],
    )(a0, d0, x, W_l, W_r, b_l[None, :])

    gmax, gsum, gcnt = pl.pallas_call(
        _tc_b_body,
        grid=(_NB,),
        in_specs=[pl.BlockSpec((_BR, _H), lambda i: (i, 0)),
                  col1_spec, vec_spec, vec_spec, vec_spec, vec_spec],
        out_specs=[pl.BlockSpec((_G, _H), lambda i: (0, 0)),
                   pl.BlockSpec((_G, _H), lambda i: (0, 0)),
                   pl.BlockSpec((_G, 1), lambda i: (0, 0))],
        out_shape=[jax.ShapeDtypeStruct((_G, _H), jnp.float32),
                   jax.ShapeDtypeStruct((_G, _H), jnp.float32),
                   jax.ShapeDtypeStruct((_G, 1), jnp.float32)],
    )(h_raw, batch[:, None], s1, s2, gamma1[None, :], beta1[None, :])

    logits = pl.pallas_call(
        _tc_c_body,
        in_specs=[pl.BlockSpec((_G, _H), lambda: (0, 0)),
                  pl.BlockSpec((_G, _H), lambda: (0, 0)),
                  pl.BlockSpec((_G, 1), lambda: (0, 0)),
                  pl.BlockSpec((_H, _H), lambda: (0, 0)),
                  pl.BlockSpec((_H, _H), lambda: (0, 0)),
                  pl.BlockSpec((1, _H), lambda: (0, 0)),
                  pl.BlockSpec((1, _H), lambda: (0, 0)),
                  pl.BlockSpec((1, _H), lambda: (0, 0)),
                  pl.BlockSpec((_H, _C), lambda: (0, 0)),
                  pl.BlockSpec((1, _C), lambda: (0, 0))],
        out_specs=pl.BlockSpec((_G, _C), lambda: (0, 0)),
        out_shape=jax.ShapeDtypeStruct((_G, _C), jnp.float32),
    )(gmax, gsum, gcnt, fc1_W[:_H], fc1_W[_H:], fc1_b[None, :],
      gamma4[None, :], beta4[None, :], fc2_W, fc2_b[None, :])

    return logits


# speedup vs baseline: 1.5741x; 1.0047x over previous
import jax
import jax.numpy as jnp
from jax import lax
from jax.experimental import pallas as pl
from jax.experimental.pallas import tpu as pltpu
from jax.experimental.pallas import tpu_sc as plsc

_N = 10000
_E = 320000
_F = 128
_H = 512
_G = 64
_C = 40

_NC = 2
_NS = 16
_NW = _NC * _NS
_L = 16

_K = 128
_NCH = 157
_EPC = _NCH * _K
_EP = _EPC * _NS
_NP = 10240
_ZR = 80
_RPC = _NP // _ZR // _NS


def _sc_body(xt_hbm, srct_hbm, dst_hbm, zero_hbm, agg_out,
             srcbuf, dstbuf, rows, zrow, agg_sh, gsem):
    cid = lax.axis_index("c")
    sid = lax.axis_index("s")

    pltpu.sync_copy(zero_hbm, zrow)

    def zero_region(k, carry):
        ro = pl.multiple_of((sid * _RPC + k) * _ZR, 8)
        pltpu.sync_copy(zrow, agg_sh.at[pl.ds(ro, _ZR)])
        return carry

    lax.fori_loop(0, _RPC, zero_region, 0)
    plsc.subcore_barrier()

    def chunk(c, carry):
        base = pl.multiple_of(sid * _EPC + c * _K, 128)
        pltpu.sync_copy(srct_hbm.at[cid].at[pl.ds(base, _K)], srcbuf)
        pltpu.sync_copy(dst_hbm.at[pl.ds(base, _K)], dstbuf)
        pltpu.async_copy(xt_hbm.at[srcbuf], rows, gsem).wait()
        pltpu.sync_copy(rows, agg_sh.at[dstbuf], add=True)
        return carry

    lax.fori_loop(0, _NCH, chunk, 0)
    plsc.subcore_barrier()

    def out_region(k, carry):
        ro = pl.multiple_of((sid * _RPC + k) * _ZR, 8)
        pltpu.sync_copy(agg_sh.at[pl.ds(ro, _ZR)],
                        agg_out.at[cid].at[pl.ds(ro, _ZR)])
        return carry

    lax.fori_loop(0, _RPC, out_region, 0)


def _build_sc_aggregate():
    mesh = plsc.VectorSubcoreMesh(
        core_axis_name="c", subcore_axis_name="s",
        num_cores=_NC, num_subcores=_NS)
    return pl.kernel(
        _sc_body,
        out_type=jax.ShapeDtypeStruct((_NC, _NP, _F), jnp.float32),
        mesh=mesh,
        scratch_types=[
            pltpu.VMEM((_K,), jnp.int32),
            pltpu.VMEM((_K,), jnp.int32),
            pltpu.VMEM((_K, _F), jnp.float32),
            pltpu.VMEM((_ZR, _F), jnp.float32),
            pltpu.VMEM_SHARED((_NP, _F), jnp.float32),
            pltpu.SemaphoreType.DMA,
        ],
    )


def _sc_aggregate(x, src, dst):
    xt = jnp.concatenate(
        [x, jnp.ones_like(x), jnp.zeros((8, _F), jnp.float32)], axis=0)
    pad = jnp.full((_EP - _E,), 2 * _N, jnp.int32)
    srct = jnp.stack([jnp.concatenate([src, pad]),
                      jnp.concatenate([src + _N, pad])])
    dstp = jnp.concatenate([dst, jnp.full((_EP - _E,), _N, jnp.int32)])
    zero_blk = jnp.zeros((_ZR, _F), jnp.float32)
    return _build_sc_aggregate()(xt, srct, dstp, zero_blk)


_BR = 400
_NB = _N // _BR


def _tc_ab_body(a0, d0, x, wl, wr, bl, b_in, g1, be1,
                gmax, gsum, gcnt, h_scr, s1, s2):
    i = pl.program_id(0)

    @pl.when(i < _NB)
    def _():
        d = jnp.maximum(d0[...], 1.0)
        mean = a0[...] / d
        h = (jnp.dot(mean, wl[...], preferred_element_type=jnp.float32)
             + jnp.dot(x[...], wr[...], preferred_element_type=jnp.float32)
             + bl[...])
        h_scr[pl.ds(i * _BR, _BR), :] = h

        @pl.when(i == 0)
        def _():
            s1[...] = jnp.zeros_like(s1)
            s2[...] = jnp.zeros_like(s2)

        s1[...] += jnp.sum(h, axis=0, keepdims=True)
        s2[...] += jnp.sum(h * h, axis=0, keepdims=True)

    @pl.when(i >= _NB)
    def _():
        j = i - _NB
        mu = s1[...] * (1.0 / _N)
        var = s2[...] * (1.0 / _N) - mu * mu
        rstd = lax.rsqrt(var + 1e-5)
        h = h_scr[pl.ds(j * _BR, _BR), :]
        h = (h - mu) * rstd * g1[...] + be1[...]
        h = jnp.maximum(h, 0.0)

        bcol = b_in[...]
        iota_g = lax.broadcasted_iota(jnp.int32, (1, _G), 1)
        oh = (bcol == iota_g).astype(jnp.float32)

        @pl.when(j == 0)
        def _():
            gmax[...] = jnp.full_like(gmax, -1e30)
            gsum[...] = jnp.zeros_like(gsum)
            gcnt[...] = jnp.zeros_like(gcnt)

        gsum[...] += lax.dot_general(oh, h, (((0,), (0,)), ((), ())),
                                     preferred_element_type=jnp.float32)
        ones_col = jnp.ones((_BR, 1), jnp.float32)
        gcnt[...] += lax.dot_general(oh, ones_col, (((0,), (0,)), ((), ())),
                                     preferred_element_type=jnp.float32)

        glo = jnp.min(bcol)
        ghi = jnp.max(bcol)
        row_ids = lax.broadcasted_iota(jnp.int32, (_G, 1), 0)
        acc = gmax[...]

        def mbody(g, acc):
            m = jnp.max(jnp.where(bcol == g, h, -1e30), axis=0, keepdims=True)
            return jnp.where(row_ids == g, jnp.maximum(acc, m), acc)

        gmax[...] = lax.fori_loop(glo, ghi + 1, mbody, acc)


def _tc_c_body(gmax, gsum, gcnt, w1a, w1b, b1, g4, be4, w2, b2, out):
    mx = gmax[...]
    mx = jnp.where(mx > -1e29, mx, 0.0)
    mean = gsum[...] / jnp.maximum(gcnt[...], 1.0)
    z = (jnp.dot(mx, w1a[...], preferred_element_type=jnp.float32)
         + jnp.dot(mean, w1b[...], preferred_element_type=jnp.float32)
         + b1[...])
    mu = jnp.mean(z, axis=0, keepdims=True)
    var = jnp.mean(z * z, axis=0, keepdims=True) - mu * mu
    z = (z - mu) * lax.rsqrt(var + 1e-5) * g4[...] + be4[...]
    z = jnp.maximum(z, 0.0)
    out[...] = jnp.dot(z, w2[...], preferred_element_type=jnp.float32) + b2[...]


def kernel(x, edge_index, batch, W_l, b_l, W_r, gamma1, beta1,
           fc1_W, fc1_b, gamma4, beta4, fc2_W, fc2_b):
    src = edge_index[0]
    dst = edge_index[1]

    aggdeg = _sc_aggregate(x, src, dst)
    a0 = aggdeg[0, :_N]
    d0 = aggdeg[1, :_N, 0:1]

    row_spec = pl.BlockSpec((_BR, _F), lambda i: (i, 0))
    col1_spec = pl.BlockSpec((_BR, 1), lambda i: (i, 0))
    w_spec = pl.BlockSpec((_F, _H), lambda i: (0, 0))
    vec_spec = pl.BlockSpec((1, _H), lambda i: (0, 0))

    half = lambda i: jnp.where(i < _NB, i, i - _NB)
    rowh_spec = pl.BlockSpec((_BR, _F), lambda i: (half(i), 0))
    col1h_spec = pl.BlockSpec((_BR, 1), lambda i: (half(i), 0))

    gmax, gsum, gcnt = pl.pallas_call(
        _tc_ab_body,
        grid=(2 * _NB,),
        in_specs=[rowh_spec, col1h_spec, rowh_spec,
                  w_spec, w_spec, vec_spec,
                  col1h_spec, vec_spec, vec_spec],
        out_specs=[pl.BlockSpec((_G, _H), lambda i: (0, 0)),
                   pl.BlockSpec((_G, _H), lambda i: (0, 0)),
                   pl.BlockSpec((_G, 1), lambda i: (0, 0))],
        out_shape=[jax.ShapeDtypeStruct((_G, _H), jnp.float32),
                   jax.ShapeDtypeStruct((_G, _H), jnp.float32),
                   jax.ShapeDtypeStruct((_G, 1), jnp.float32)],
        scratch_shapes=[pltpu.VMEM((_N, _H), jnp.float32),
                        pltpu.VMEM((1, _H), jnp.float32),
                        pltpu.VMEM((1, _H), jnp.float32)],
    )(a0, d0, x, W_l, W_r, b_l[None, :],
      batch[:, None], gamma1[None, :], beta1[None, :])

    logits = pl.pallas_call(
        _tc_c_body,
        in_specs=[pl.BlockSpec((_G, _H), lambda: (0, 0)),
                  pl.BlockSpec((_G, _H), lambda: (0, 0)),
                  pl.BlockSpec((_G, 1), lambda: (0, 0)),
                  pl.BlockSpec((_H, _H), lambda: (0, 0)),
                  pl.BlockSpec((_H, _H), lambda: (0, 0)),
                  pl.BlockSpec((1, _H), lambda: (0, 0)),
                  pl.BlockSpec((1, _H), lambda: (0, 0)),
                  pl.BlockSpec((1, _H), lambda: (0, 0)),
                  pl.BlockSpec((_H, _C), lambda: (0, 0)),
                  pl.BlockSpec((1, _C), lambda: (0, 0))],
        out_specs=pl.BlockSpec((_G, _C), lambda: (0, 0)),
        out_shape=jax.ShapeDtypeStruct((_G, _C), jnp.float32),
    )(gmax, gsum, gcnt, fc1_W[:_H], fc1_W[_H:], fc1_b[None, :],
      gamma4[None, :], beta4[None, :], fc2_W, fc2_b[None, :])

    return logits
